# baseline (speedup 1.0000x reference)
import functools

import jax
import jax.numpy as jnp
from jax import lax
from jax.experimental import pallas as pl
from jax.experimental.pallas import tpu as pltpu
from jax.experimental.pallas import tpu_sc as plsc

_B, _H, _W = 16, 512, 512
_CARD = 50
_N = _B * _CARD * 2
_PER_TILE = 64
_ACTIVE = _N // _PER_TILE
_LANES = 16


def _sc_gather(x2d, packed):
    mesh = plsc.VectorSubcoreMesh(core_axis_name="c", subcore_axis_name="s")
    info = plsc.get_sparse_core_info()
    num_cores = info.num_cores

    @functools.partial(
        pl.kernel,
        mesh=mesh,
        out_type=jax.ShapeDtypeStruct((_N,), jnp.float32),
        scratch_types=[
            pltpu.VMEM((_PER_TILE,), jnp.int32),
            pltpu.VMEM((_PER_TILE,), jnp.int32),
            pltpu.VMEM((_PER_TILE, _W), jnp.float32),
            pltpu.VMEM((_PER_TILE,), jnp.float32),
            pltpu.SemaphoreType.DMA,
        ],
    )
    def body(x_hbm, p_hbm, out_hbm, p_v, ridx_v, rows_v, vals_v, sem):
        wid = lax.axis_index("s") * num_cores + lax.axis_index("c")

        @pl.when(wid < _ACTIVE)
        def _():
            base = wid * _PER_TILE
            pltpu.sync_copy(p_hbm.at[pl.ds(base, _PER_TILE)], p_v)
            lanes = lax.iota(jnp.int32, _LANES)
            for g in range(_PER_TILE // _LANES):
                s = pl.ds(g * _LANES, _LANES)
                ridx_v[s] = p_v[s] >> 10
            pltpu.async_copy(x_hbm.at[ridx_v], rows_v, sem).wait()
            for g in range(_PER_TILE // _LANES):
                s = pl.ds(g * _LANES, _LANES)
                w16 = p_v[s]
                acc = jnp.zeros((_LANES,), jnp.float32)
                for l in range(_LANES):
                    j = g * _LANES + l
                    c = (w16[l] >> 1) & jnp.int32(_W - 1)
                    c16 = pl.multiple_of(c & ~15, 16)
                    lane = c & 15
                    v16 = rows_v[j, pl.ds(c16, _LANES)]
                    val = v16[0]
                    for k in range(1, _LANES):
                        val = jnp.where(lane == k, v16[k], val)
                    acc = jnp.where(lanes == l, val, acc)
                vals_v[s] = jnp.where((w16 & 1) != 0, jnp.float32(0.0), acc)
            pltpu.sync_copy(vals_v, out_hbm.at[pl.ds(base, _PER_TILE)])

    return body(x2d, packed)


def kernel(x, ids0, ids1, ids_mask):
    x2d = x.reshape(_B * _H, _W)
    packed = (
        (ids0 << 10) | (ids1 << 1) | ids_mask.astype(jnp.int32)
    ).reshape(_N)
    flat = _sc_gather(x2d, packed)
    return flat.reshape(_B, _CARD, 2)


# speedup vs baseline: 1.0103x; 1.0103x over previous
import functools

import jax
import jax.numpy as jnp
from jax import lax
from jax.experimental import pallas as pl
from jax.experimental.pallas import tpu as pltpu
from jax.experimental.pallas import tpu_sc as plsc

_B, _H, _W = 16, 512, 512
_CARD = 50
_N = _B * _CARD * 2
_PER_TILE = 64
_ACTIVE = _N // _PER_TILE
_LANES = 16


def _sc_gather(x2d, packed):
    mesh = plsc.VectorSubcoreMesh(core_axis_name="c", subcore_axis_name="s")
    info = plsc.get_sparse_core_info()
    num_cores = info.num_cores

    @functools.partial(
        pl.kernel,
        mesh=mesh,
        out_type=jax.ShapeDtypeStruct((_N,), jnp.float32),
        scratch_types=[
            pltpu.VMEM((_PER_TILE,), jnp.int32),
            pltpu.VMEM((_PER_TILE,), jnp.int32),
            pltpu.VMEM((_PER_TILE, _W), jnp.float32),
            pltpu.VMEM((_PER_TILE,), jnp.float32),
            pltpu.SemaphoreType.DMA,
        ],
    )
    def body(x_hbm, p_hbm, out_hbm, p_v, ridx_v, rows_v, vals_v, sem):
        wid = lax.axis_index("s") * num_cores + lax.axis_index("c")

        @pl.when(wid < _ACTIVE)
        def _():
            base = wid * _PER_TILE
            pltpu.sync_copy(p_hbm.at[pl.ds(base, _PER_TILE)], p_v)
            lanes = lax.iota(jnp.int32, _LANES)
            for g in range(_PER_TILE // _LANES):
                s = pl.ds(g * _LANES, _LANES)
                ridx_v[s] = p_v[s] >> 10
            pltpu.async_copy(x_hbm.at[ridx_v], rows_v, sem).wait()
            for g in range(_PER_TILE // _LANES):
                s = pl.ds(g * _LANES, _LANES)
                w16 = p_v[s]
                acc = jnp.zeros((_LANES,), jnp.float32)
                for l in range(_LANES):
                    j = g * _LANES + l
                    c = (w16[l] >> 1) & jnp.int32(_W - 1)
                    c16 = pl.multiple_of(c & ~15, 16)
                    lane = c & 15
                    v16 = rows_v[j, pl.ds(c16, _LANES)]
                    tv = jnp.where(lanes == lane, v16, jnp.float32(0.0))
                    parts = [tv[k] for k in range(_LANES)]
                    while len(parts) > 1:
                        parts = [parts[i] + parts[i + 1]
                                 for i in range(0, len(parts), 2)]
                    acc = jnp.where(lanes == l, parts[0], acc)
                vals_v[s] = jnp.where((w16 & 1) != 0, jnp.float32(0.0), acc)
            pltpu.sync_copy(vals_v, out_hbm.at[pl.ds(base, _PER_TILE)])

    return body(x2d, packed)


def kernel(x, ids0, ids1, ids_mask):
    x2d = x.reshape(_B * _H, _W)
    packed = (
        (ids0 << 10) | (ids1 << 1) | ids_mask.astype(jnp.int32)
    ).reshape(_N)
    flat = _sc_gather(x2d, packed)
    return flat.reshape(_B, _CARD, 2)


# speedup vs baseline: 1.5300x; 1.5144x over previous
import functools

import jax
import jax.numpy as jnp
from jax import lax
from jax.experimental import pallas as pl
from jax.experimental.pallas import tpu as pltpu
from jax.experimental.pallas import tpu_sc as plsc

_B, _H, _W = 16, 512, 512
_CARD = 50
_N = _B * _CARD * 2
_PER_TILE = 64
_ACTIVE = _N // _PER_TILE
_LANES = 16
_LR = 128


def _sc_gather(xlr, packed):
    mesh = plsc.VectorSubcoreMesh(core_axis_name="c", subcore_axis_name="s")
    info = plsc.get_sparse_core_info()
    num_cores = info.num_cores

    @functools.partial(
        pl.kernel,
        mesh=mesh,
        out_type=(
            jax.ShapeDtypeStruct((_N,), jnp.float32),
            jax.ShapeDtypeStruct((_N * _LR,), jnp.float32),
        ),
        scratch_types=[
            pltpu.VMEM((_PER_TILE,), jnp.int32),
            pltpu.VMEM((_PER_TILE,), jnp.int32),
            pltpu.VMEM((_PER_TILE,), jnp.int32),
            pltpu.VMEM((_PER_TILE, _LR), jnp.float32),
            pltpu.VMEM((_PER_TILE,), jnp.float32),
            pltpu.SemaphoreType.DMA,
            pltpu.SemaphoreType.DMA,
        ],
    )
    def body(x_hbm, p_hbm, out_hbm, stage_hbm,
             p_v, ridx_v, eidx_v, lr_v, vals_v, sem, wsem):
        wid = lax.axis_index("s") * num_cores + lax.axis_index("c")

        @pl.when(wid < _ACTIVE)
        def _():
            base = wid * _PER_TILE
            pltpu.sync_copy(p_hbm.at[pl.ds(base, _PER_TILE)], p_v)
            lanes = lax.iota(jnp.int32, _LANES)
            for g in range(_PER_TILE // _LANES):
                s = pl.ds(g * _LANES, _LANES)
                w = p_v[s]
                r = w >> 10
                c = (w >> 1) & jnp.int32(_W - 1)
                ridx_v[s] = ((r >> 3) * 4 + (c >> 7)) * 8 + (r & 7)
                j = jnp.int32(g * _LANES) + lanes
                eidx_v[s] = (jnp.int32(base) + j) * _LR + (c & 127)
            pltpu.async_copy(x_hbm.at[ridx_v], lr_v, sem).wait()

            def _wr(j, carry):
                off = pl.multiple_of((jnp.int32(base) + j) * _LR, _LR)
                pltpu.async_copy(lr_v.at[j], stage_hbm.at[pl.ds(off, _LR)],
                                 wsem)
                return carry

            lax.fori_loop(0, _PER_TILE, _wr, 0)

            def _dr(j, carry):
                off = pl.multiple_of((jnp.int32(base) + j) * _LR, _LR)
                pltpu.make_async_copy(
                    lr_v.at[j], stage_hbm.at[pl.ds(off, _LR)], wsem
                ).wait()
                return carry

            lax.fori_loop(0, _PER_TILE, _dr, 0)

            pltpu.async_copy(stage_hbm.at[eidx_v], vals_v, sem).wait()
            for g in range(_PER_TILE // _LANES):
                s = pl.ds(g * _LANES, _LANES)
                vals_v[s] = jnp.where((p_v[s] & 1) != 0, jnp.float32(0.0),
                                      vals_v[s])
            pltpu.sync_copy(vals_v, out_hbm.at[pl.ds(base, _PER_TILE)])

    return body(xlr, packed)


def kernel(x, ids0, ids1, ids_mask):
    xlr = (
        x.reshape(1024, 8, 4, 128).transpose(0, 2, 1, 3).reshape(32768, 128)
    )
    packed = (
        (ids0 << 10) | (ids1 << 1) | ids_mask.astype(jnp.int32)
    ).reshape(_N)
    flat, _ = _sc_gather(xlr, packed)
    return flat.reshape(_B, _CARD, 2)


# speedup vs baseline: 1.5725x; 1.0278x over previous
import functools

import jax
import jax.numpy as jnp
from jax import lax
from jax.experimental import pallas as pl
from jax.experimental.pallas import tpu as pltpu
from jax.experimental.pallas import tpu_sc as plsc

_B, _H, _W = 16, 512, 512
_CARD = 50
_N = _B * _CARD * 2
_PER_TILE = 64
_ACTIVE = _N // _PER_TILE
_LANES = 16
_LR = 128


def _sc_gather(xlr, packed):
    mesh = plsc.VectorSubcoreMesh(core_axis_name="c", subcore_axis_name="s")
    info = plsc.get_sparse_core_info()
    num_cores = info.num_cores

    @functools.partial(
        pl.kernel,
        mesh=mesh,
        out_type=jax.ShapeDtypeStruct((_N,), jnp.float32),
        scratch_types=[
            pltpu.VMEM((_PER_TILE,), jnp.int32),
            pltpu.VMEM((_PER_TILE,), jnp.int32),
            pltpu.VMEM((_PER_TILE,), jnp.int32),
            pltpu.VMEM((_PER_TILE, _LR), jnp.float32),
            pltpu.VMEM((_PER_TILE,), jnp.float32),
            pltpu.VMEM_SHARED((16 * _PER_TILE * _LR,), jnp.float32),
            pltpu.SemaphoreType.DMA,
            pltpu.SemaphoreType.DMA,
        ],
    )
    def body(x_hbm, p_hbm, out_hbm,
             p_v, ridx_v, eidx_v, lr_v, vals_v, stage_sp, sem, wsem):
        wid = lax.axis_index("s") * num_cores + lax.axis_index("c")

        sub = lax.axis_index("s")

        @pl.when(wid < _ACTIVE)
        def _():
            base = wid * _PER_TILE
            sbase = sub * jnp.int32(_PER_TILE * _LR)
            pltpu.sync_copy(p_hbm.at[pl.ds(base, _PER_TILE)], p_v)
            lanes = lax.iota(jnp.int32, _LANES)
            for g in range(_PER_TILE // _LANES):
                s = pl.ds(g * _LANES, _LANES)
                w = p_v[s]
                r = w >> 10
                c = (w >> 1) & jnp.int32(_W - 1)
                ridx_v[s] = ((r >> 3) * 4 + (c >> 7)) * 8 + (r & 7)
                j = jnp.int32(g * _LANES) + lanes
                eidx_v[s] = sbase + j * _LR + (c & 127)
            pltpu.async_copy(x_hbm.at[ridx_v], lr_v, sem).wait()

            def _wr(j, carry):
                off = pl.multiple_of(sbase + j * _LR, _LR)
                pltpu.async_copy(lr_v.at[j], stage_sp.at[pl.ds(off, _LR)],
                                 wsem)
                return carry

            lax.fori_loop(0, _PER_TILE, _wr, 0)

            def _dr(j, carry):
                off = pl.multiple_of(sbase + j * _LR, _LR)
                pltpu.make_async_copy(
                    lr_v.at[j], stage_sp.at[pl.ds(off, _LR)], wsem
                ).wait()
                return carry

            lax.fori_loop(0, _PER_TILE, _dr, 0)

            pltpu.async_copy(stage_sp.at[eidx_v], vals_v, sem).wait()
            for g in range(_PER_TILE // _LANES):
                s = pl.ds(g * _LANES, _LANES)
                vals_v[s] = jnp.where((p_v[s] & 1) != 0, jnp.float32(0.0),
                                      vals_v[s])
            pltpu.sync_copy(vals_v, out_hbm.at[pl.ds(base, _PER_TILE)])

    return body(xlr, packed)


def kernel(x, ids0, ids1, ids_mask):
    xlr = (
        x.reshape(1024, 8, 4, 128).transpose(0, 2, 1, 3).reshape(32768, 128)
    )
    packed = (
        (ids0 << 10) | (ids1 << 1) | ids_mask.astype(jnp.int32)
    ).reshape(_N)
    flat = _sc_gather(xlr, packed)
    return flat.reshape(_B, _CARD, 2)
